# baseline (speedup 1.0000x reference)
import functools

import jax
import jax.numpy as jnp
from jax import lax
from jax.experimental import pallas as pl
from jax.experimental.pallas import tpu as pltpu
from jax.experimental.pallas import tpu_sc as plsc

NORM_PERCENT = 50
CB = 32
SC_LANES = 16


def _sumsq_body(x_ref, out_ref):
    xb = x_ref[...]
    out_ref[...] = jnp.sum(xb * xb, axis=1).reshape(1, 1, -1)


def _sc_rot_gather(v, idx):
    return lax.gather(
        v, idx[:, None],
        lax.GatherDimensionNumbers(
            offset_dims=(), collapsed_slice_dims=(0,), start_index_map=(0,)),
        slice_sizes=(1,),
        mode=lax.GatherScatterMode.PROMISE_IN_BOUNDS,
    )


def _sc_rot_gather(v, idx):
    return lax.gather(
        v, idx[:, None],
        lax.GatherDimensionNumbers(
            offset_dims=(), collapsed_slice_dims=(0,), start_index_map=(0,)),
        slice_sizes=(1,),
        mode=lax.GatherScatterMode.PROMISE_IN_BOUNDS,
    )


def _plan_sc_body(k, s2_hbm, keep_hbm, s2_v, keep_v):
    C = s2_hbm.shape[0]
    nchunks = C // SC_LANES
    w = lax.axis_index("s") * 2 + lax.axis_index("c")
    pltpu.sync_copy(s2_hbm, s2_v)

    lanes = lax.iota(jnp.int32, SC_LANES)
    base1 = w * SC_LANES
    ch2 = jnp.minimum(w + (nchunks - 32), nchunks - 1)
    base2 = ch2 * SC_LANES
    mine1 = s2_v[pl.ds(base1, SC_LANES)]
    mine2 = s2_v[pl.ds(base2, SC_LANES)]
    myidx1 = lanes + base1
    myidx2 = lanes + base2
    zero = jnp.zeros((SC_LANES,), jnp.int32)

    def body(r, carry):
        c1, c2 = carry
        rot = r & (SC_LANES - 1)
        bj = r - rot
        sjv = s2_v[pl.ds(bj, SC_LANES)]
        idx = (lanes + rot) & (SC_LANES - 1)
        g = _sc_rot_gather(sjv, idx)
        jvec = bj + idx
        b1 = (g > mine1) | ((g == mine1) & (jvec < myidx1))
        b2 = (g > mine2) | ((g == mine2) & (jvec < myidx2))
        return c1 + b1.astype(jnp.int32), c2 + b2.astype(jnp.int32)

    cnt1, cnt2 = lax.fori_loop(0, C, body, (zero, zero))

    keep_v[...] = (cnt1 < k).astype(jnp.int32)
    pltpu.sync_copy(keep_v, keep_hbm.at[pl.ds(base1, SC_LANES)])

    @pl.when(w >= 64 - nchunks)
    def _():
        keep_v[...] = (cnt2 < k).astype(jnp.int32)
        pltpu.sync_copy(keep_v, keep_hbm.at[pl.ds(base2, SC_LANES)])


def _mul_body(plan_ref, x_hbm, o_hbm, xbuf, zbuf, rsems, wsems):
    b = pl.program_id(0)
    nb = pl.num_programs(0)

    def rd(c, sl, ch):
        return pltpu.make_async_copy(
            x_hbm.at[pl.ds(c, 1)], xbuf.at[sl, pl.ds(ch, 1)], rsems.at[sl]
        )

    def reads(bb, sl, action):
        for ch in range(CB):
            c = bb * CB + ch

            @pl.when(plan_ref[0, c] == 1)
            def _():
                action(rd(c, sl, ch))

    def writes(bb, sl, action):
        for ch in range(CB):
            c = bb * CB + ch

            @pl.when(plan_ref[0, c] == 1)
            def _():
                action(pltpu.make_async_copy(
                    xbuf.at[sl, pl.ds(ch, 1)], o_hbm.at[pl.ds(c, 1)],
                    wsems.at[sl],
                ))

            @pl.when(plan_ref[0, c] == 0)
            def _():
                action(pltpu.make_async_copy(
                    zbuf, o_hbm.at[pl.ds(c, 1)], wsems.at[sl],
                ))

    @pl.when(b == 0)
    def _():
        zbuf[...] = jnp.zeros_like(zbuf)
        reads(0, 0, lambda cp: cp.start())

    @pl.when(b > 0)
    def _():
        writes(b - 1, (b - 1) % 2, lambda cp: cp.wait())

    @pl.when(b + 1 < nb)
    def _():
        reads(b + 1, (b + 1) % 2, lambda cp: cp.start())

    reads(b, b % 2, lambda cp: cp.wait())
    writes(b, b % 2, lambda cp: cp.start())

    @pl.when(b == nb - 1)
    def _():
        writes(b, b % 2, lambda cp: cp.wait())


def kernel(input):
    x = input
    _, C, H, W = x.shape
    k = int(float(NORM_PERCENT) / 100.0 * float(C))
    if k <= 0 or k >= C:
        k = C
    HW = H * W
    nblk = C // CB

    x2 = x.reshape(C, HW)

    sumsq = pl.pallas_call(
        _sumsq_body,
        grid=(nblk,),
        in_specs=[pl.BlockSpec((CB, HW), lambda i: (i, 0))],
        out_specs=pl.BlockSpec((1, 1, CB), lambda i: (i, 0, 0)),
        out_shape=jax.ShapeDtypeStruct((nblk, 1, CB), jnp.float32),
    )(x2)

    plan1d = pl.kernel(
        functools.partial(_plan_sc_body, k),
        out_type=jax.ShapeDtypeStruct((C,), jnp.int32),
        scratch_types=[
            pltpu.VMEM((C,), jnp.float32),
            pltpu.VMEM((SC_LANES,), jnp.int32),
        ],
        mesh=plsc.VectorSubcoreMesh(core_axis_name="c", subcore_axis_name="s"),
        compiler_params=pltpu.CompilerParams(needs_layout_passes=False),
    )(sumsq.reshape(C))
    plan = plan1d.reshape(1, C)

    grid_spec = pltpu.PrefetchScalarGridSpec(
        num_scalar_prefetch=1,
        grid=(nblk,),
        in_specs=[pl.BlockSpec(memory_space=pl.ANY)],
        out_specs=pl.BlockSpec(memory_space=pl.ANY),
        scratch_shapes=[
            pltpu.VMEM((2, CB, HW), jnp.float32),
            pltpu.VMEM((1, HW), jnp.float32),
            pltpu.SemaphoreType.DMA((2,)),
            pltpu.SemaphoreType.DMA((2,)),
        ],
    )
    out = pl.pallas_call(
        _mul_body,
        grid_spec=grid_spec,
        out_shape=jax.ShapeDtypeStruct((C, HW), jnp.float32),
    )(plan, x2)

    return out.reshape(x.shape)
